# baseline (speedup 1.0000x reference)
import dataclasses
import functools

import jax
import jax.numpy as jnp
from jax import lax
from jax.experimental import pallas as pl
from jax.experimental.pallas import tpu as pltpu
from jax.experimental.pallas import tpu_sc as plsc

_B = 16384
_HIDDEN = 128
_ITEM_NUM = 100000
_MAX_SLIP = 0.4
_MAX_GUESS = 0.4
_T = 50.0

_NC = 2
_NS = 16
_NW = _NC * _NS
_B_PER_W = _B // _NW
_T_PER_W = 4096
_T_PAD = _NW * _T_PER_W
_SG_ROWS = _T_PAD // 128
_QTR = _SG_ROWS // 4

_CP = dataclasses.replace(pltpu.CompilerParams(), needs_layout_passes=False)


def _sc_build_sg(slip_flat, guess_flat):
    mesh = plsc.VectorSubcoreMesh(core_axis_name="c", subcore_axis_name="s")

    @functools.partial(
        pl.kernel,
        out_type=jax.ShapeDtypeStruct((_SG_ROWS, 128), jnp.int32),
        mesh=mesh,
        scratch_types=[
            pltpu.VMEM((_T_PER_W,), jnp.float32),
            pltpu.VMEM((_T_PER_W,), jnp.float32),
            pltpu.VMEM((_T_PER_W // 128, 128), jnp.int32),
        ],
        compiler_params=_CP,
    )
    def build_kernel(slip_hbm, guess_hbm, sg_out, s_v, g_v, b_v):
        wid = lax.axis_index("s") * _NC + lax.axis_index("c")
        tbase = pl.multiple_of(wid * _T_PER_W, 8)
        pltpu.sync_copy(slip_hbm.at[pl.ds(tbase, _T_PER_W)], s_v)
        pltpu.sync_copy(guess_hbm.at[pl.ds(tbase, _T_PER_W)], g_v)
        i16 = lax.iota(jnp.int32, 16)

        @pl.loop(0, _T_PER_W // 16)
        def _(j):
            vs = s_v[pl.ds(j * 16, 16)]
            vg = g_v[pl.ds(j * 16, 16)]
            packed = plsc.bitcast(
                plsc.pack(vs, vg, format=plsc.PackFormat.INTERLEAVED),
                jnp.int32)
            p0 = j * 16
            rows = (i16 * 0) + (p0 // 128)
            cols = (p0 % 128) + i16
            plsc.store_scatter(b_v, [rows, cols], packed)

        obase = pl.multiple_of(wid * (_T_PER_W // 128), 8)
        pltpu.sync_copy(b_v, sg_out.at[pl.ds(obase, _T_PER_W // 128), :])

    return build_kernel(slip_flat, guess_flat)


def _sc_gather(user, item, theta_table, sg_packed):
    mesh = plsc.VectorSubcoreMesh(core_axis_name="c", subcore_axis_name="s")
    halfw = _B_PER_W // 2

    @functools.partial(
        pl.kernel,
        out_type=(
            jax.ShapeDtypeStruct((_B, _HIDDEN), jnp.float32),
            jax.ShapeDtypeStruct((_B,), jnp.float32),
            jax.ShapeDtypeStruct((_B,), jnp.float32),
        ),
        mesh=mesh,
        scratch_types=[
            pltpu.VMEM((_B_PER_W,), jnp.int32),
            pltpu.VMEM((_B_PER_W,), jnp.int32),
            pltpu.VMEM((halfw, _HIDDEN), jnp.float32),
            pltpu.VMEM((halfw, _HIDDEN), jnp.float32),
            pltpu.VMEM((_QTR, 128), jnp.int32),
            pltpu.VMEM((_B_PER_W,), jnp.float32),
            pltpu.VMEM((_B_PER_W,), jnp.float32),
            pltpu.SemaphoreType.DMA,
        ],
        compiler_params=_CP,
    )
    def gather_kernel(user_hbm, item_hbm, theta_hbm, sg_hbm,
                      theta_out, slip_out, guess_out,
                      uidx_v, iidx_v, rows_a, rows_b, tb_v, s_v, g_v, sem_t):
        wid = lax.axis_index("s") * _NC + lax.axis_index("c")
        base = wid * _B_PER_W
        pltpu.sync_copy(user_hbm.at[pl.ds(base, _B_PER_W)], uidx_v)
        pltpu.sync_copy(item_hbm.at[pl.ds(base, _B_PER_W)], iidx_v)
        cp_a = pltpu.async_copy(
            theta_hbm.at[uidx_v.at[pl.ds(0, halfw)]], rows_a, sem_t)
        cp_b = pltpu.async_copy(
            theta_hbm.at[uidx_v.at[pl.ds(halfw, halfw)]], rows_b, sem_t)
        i16 = lax.iota(jnp.int32, 16)
        del i16

        @pl.loop(0, 4)
        def _(q):
            qbase = pl.multiple_of(q * _QTR, 8)
            pltpu.sync_copy(sg_hbm.at[pl.ds(qbase, _QTR), :], tb_v)

            @pl.loop(0, _B_PER_W // 16)
            def _(k):
                idx = iidx_v[pl.ds(k * 16, 16)]
                r = lax.shift_right_logical(idx, 7) - q * _QTR
                lane = lax.bitwise_and(idx, 127)
                in_q = (r >= 0) & (r < _QTR)
                rc = lax.max(lax.min(r, _QTR - 1), 0)
                w = plsc.load_gather(tb_v, [rc, lane])
                pair = plsc.unpack(
                    plsc.bitcast(w, jnp.bfloat16),
                    format=plsc.PackFormat.INTERLEAVED)
                vs = pair[0].astype(jnp.float32)
                vg = pair[1].astype(jnp.float32)
                sl = pl.ds(k * 16, 16)
                s_v[sl] = jnp.where(in_q, vs, s_v[sl])
                g_v[sl] = jnp.where(in_q, vg, g_v[sl])

        pltpu.sync_copy(s_v, slip_out.at[pl.ds(base, _B_PER_W)])
        pltpu.sync_copy(g_v, guess_out.at[pl.ds(base, _B_PER_W)])
        cp_a.wait()
        pltpu.sync_copy(rows_a, theta_out.at[pl.ds(base, halfw)])
        cp_b.wait()
        pltpu.sync_copy(rows_b, theta_out.at[pl.ds(base + halfw, halfw)])

    return gather_kernel(user, item, theta_table, sg_packed)


def _score_block(theta_ref, kn_ref, s_ref, g_ref, out_ref):
    th = theta_ref[...]
    kn = kn_ref[...]
    rows = th.shape[0]
    prod = kn * (jax.nn.sigmoid(th) - 0.5)
    p3 = prod.reshape(rows // _HIDDEN, _HIDDEN, _HIDDEN)
    n = jnp.sum(jnp.swapaxes(p3, 1, 2), axis=1).reshape(rows)
    s = jax.nn.sigmoid(n * (1.0 / _T))
    slip = jax.nn.sigmoid(s_ref[...]) * _MAX_SLIP
    guess = jax.nn.sigmoid(g_ref[...]) * _MAX_GUESS
    out_ref[...] = (1.0 - slip) * s + guess * (1.0 - s)


def kernel(user, item, knowledge, theta_table, slip_table, guess_table):
    pad = _T_PAD - _ITEM_NUM
    slip_flat = jnp.pad(slip_table.reshape(_ITEM_NUM), (0, pad))
    guess_flat = jnp.pad(guess_table.reshape(_ITEM_NUM), (0, pad))
    sg_packed = _sc_build_sg(slip_flat, guess_flat)

    theta_g, slip_g, guess_g = _sc_gather(
        user, item.astype(jnp.int32), theta_table, sg_packed)

    rows = 4096
    out = pl.pallas_call(
        _score_block,
        grid=(_B // rows,),
        in_specs=[
            pl.BlockSpec((rows, _HIDDEN), lambda i: (i, 0)),
            pl.BlockSpec((rows, _HIDDEN), lambda i: (i, 0)),
            pl.BlockSpec((rows,), lambda i: (i,)),
            pl.BlockSpec((rows,), lambda i: (i,)),
        ],
        out_specs=pl.BlockSpec((rows,), lambda i: (i,)),
        out_shape=jax.ShapeDtypeStruct((_B,), jnp.float32),
    )(theta_g, knowledge, slip_g, guess_g)
    return out


# speedup vs baseline: 1.2448x; 1.2448x over previous
import dataclasses
import functools

import jax
import jax.numpy as jnp
from jax import lax
from jax.experimental import pallas as pl
from jax.experimental.pallas import tpu as pltpu
from jax.experimental.pallas import tpu_sc as plsc

_B = 16384
_HIDDEN = 128
_ITEM_NUM = 100000
_MAX_SLIP = 0.4
_MAX_GUESS = 0.4
_T = 50.0

_NC = 2
_NS = 16
_NW = _NC * _NS
_B_PER_W = _B // _NW
_T_PER_W = 3200
_CHUNK = 400
_ROWS_PER_W = _T_PER_W * 2 // 128
_SLOT = 56
_SG_ROWS = _NW * _SLOT


def _sc_build_sg(slip_flat, guess_flat):
    mesh = plsc.VectorSubcoreMesh(core_axis_name="c", subcore_axis_name="s")

    @functools.partial(
        pl.kernel,
        out_type=jax.ShapeDtypeStruct((_SG_ROWS, 128), jnp.float32),
        mesh=mesh,
        scratch_types=[
            pltpu.VMEM((_T_PER_W,), jnp.float32),
            pltpu.VMEM((_T_PER_W,), jnp.float32),
            pltpu.VMEM((_SLOT, 128), jnp.float32),
        ],
        compiler_params=dataclasses.replace(
            pltpu.CompilerParams(), needs_layout_passes=False),
    )
    def build_kernel(slip_hbm, guess_hbm, sg_out, s_v, g_v, b_v):
        wid = lax.axis_index("s") * _NC + lax.axis_index("c")
        tbase = pl.multiple_of(wid * _T_PER_W, 8)
        pltpu.sync_copy(slip_hbm.at[pl.ds(tbase, _T_PER_W)], s_v)
        pltpu.sync_copy(guess_hbm.at[pl.ds(tbase, _T_PER_W)], g_v)
        i16 = lax.iota(jnp.int32, 16)

        @pl.loop(0, _T_PER_W // 16)
        def _(j):
            p0 = j * 32
            rows = (i16 * 0) + (p0 // 128)
            cols = (p0 % 128) + 2 * i16
            vs = s_v[pl.ds(j * 16, 16)]
            vg = g_v[pl.ds(j * 16, 16)]
            plsc.store_scatter(b_v, [rows, cols], vs)
            plsc.store_scatter(b_v, [rows, cols + 1], vg)

        obase = pl.multiple_of(wid * _SLOT, 8)
        pltpu.sync_copy(b_v, sg_out.at[pl.ds(obase, _SLOT), :])

    return build_kernel(slip_flat, guess_flat)


def _sc_gather(user, sg_row_idx, lane0, theta_table, sg_flat):
    mesh = plsc.VectorSubcoreMesh(core_axis_name="c", subcore_axis_name="s")

    @functools.partial(
        pl.kernel,
        out_type=(
            jax.ShapeDtypeStruct((_B, _HIDDEN), jnp.float32),
            jax.ShapeDtypeStruct((_B,), jnp.float32),
            jax.ShapeDtypeStruct((_B,), jnp.float32),
        ),
        mesh=mesh,
        scratch_types=[
            pltpu.VMEM((_B_PER_W,), jnp.int32),
            pltpu.VMEM((_B_PER_W, _HIDDEN), jnp.float32),
            pltpu.VMEM((_B_PER_W,), jnp.int32),
            pltpu.VMEM((_B_PER_W,), jnp.int32),
            pltpu.VMEM((_B_PER_W // 2, _HIDDEN), jnp.float32),
            pltpu.VMEM((_B_PER_W,), jnp.float32),
            pltpu.VMEM((_B_PER_W,), jnp.float32),
            pltpu.SemaphoreType.DMA,
            pltpu.SemaphoreType.DMA,
        ],
        compiler_params=dataclasses.replace(
            pltpu.CompilerParams(), needs_layout_passes=False),
    )
    def gather_kernel(user_hbm, sgi_hbm, lane_hbm, theta_hbm, sg_hbm,
                      theta_out, slip_out, guess_out,
                      uidx_v, rows_v, iidx_v, lane_v, sg_v, s_v, g_v,
                      sem_t, sem_s):
        wid = lax.axis_index("s") * _NC + lax.axis_index("c")
        base = wid * _B_PER_W
        pltpu.sync_copy(user_hbm.at[pl.ds(base, _B_PER_W)], uidx_v)
        pltpu.sync_copy(sgi_hbm.at[pl.ds(base, _B_PER_W)], iidx_v)
        pltpu.sync_copy(lane_hbm.at[pl.ds(base, _B_PER_W)], lane_v)
        cp_t = pltpu.async_copy(theta_hbm.at[uidx_v], rows_v, sem_t)
        half = _B_PER_W // 2
        i16 = lax.iota(jnp.int32, 16)

        @pl.loop(0, 2)
        def _(h):
            off = h * half
            cp_s = pltpu.async_copy(
                sg_hbm.at[iidx_v.at[pl.ds(off, half)]], sg_v, sem_s)
            cp_s.wait()

            @pl.loop(0, half // 16)
            def _(k):
                rows = k * 16 + i16
                cols = lane_v[pl.ds(off + k * 16, 16)]
                s_v[pl.ds(off + k * 16, 16)] = plsc.load_gather(
                    sg_v, [rows, cols])
                g_v[pl.ds(off + k * 16, 16)] = plsc.load_gather(
                    sg_v, [rows, cols + 1])

        pltpu.sync_copy(s_v, slip_out.at[pl.ds(base, _B_PER_W)])
        pltpu.sync_copy(g_v, guess_out.at[pl.ds(base, _B_PER_W)])
        cp_t.wait()
        pltpu.sync_copy(rows_v, theta_out.at[pl.ds(base, _B_PER_W)])

    return gather_kernel(user, sg_row_idx, lane0, theta_table, sg_flat)


def _score_block(theta_ref, kn_ref, s_ref, g_ref, out_ref):
    th = theta_ref[...]
    kn = kn_ref[...]
    rows = th.shape[0]
    prod = kn * (jax.nn.sigmoid(th) - 0.5)
    p3 = prod.reshape(rows // _HIDDEN, _HIDDEN, _HIDDEN)
    n = jnp.sum(jnp.swapaxes(p3, 1, 2), axis=1).reshape(rows)
    s = jax.nn.sigmoid(n * (1.0 / _T))
    slip = jax.nn.sigmoid(s_ref[...]) * _MAX_SLIP
    guess = jax.nn.sigmoid(g_ref[...]) * _MAX_GUESS
    out_ref[...] = (1.0 - slip) * s + guess * (1.0 - s)


def kernel(user, item, knowledge, theta_table, slip_table, guess_table):
    pad = _NW * _T_PER_W - _ITEM_NUM
    slip_flat = jnp.pad(slip_table.reshape(_ITEM_NUM), (0, pad))
    guess_flat = jnp.pad(guess_table.reshape(_ITEM_NUM), (0, pad))
    sg_flat = _sc_build_sg(slip_flat, guess_flat)

    w = item // _T_PER_W
    p = (item % _T_PER_W) * 2
    sg_row_idx = (w * _SLOT + p // 128).astype(jnp.int32)
    lane0 = (p % 128).astype(jnp.int32)

    theta_g, slip_g, guess_g = _sc_gather(
        user, sg_row_idx, lane0, theta_table, sg_flat)

    rows = 4096
    out = pl.pallas_call(
        _score_block,
        grid=(_B // rows,),
        in_specs=[
            pl.BlockSpec((rows, _HIDDEN), lambda i: (i, 0)),
            pl.BlockSpec((rows, _HIDDEN), lambda i: (i, 0)),
            pl.BlockSpec((rows,), lambda i: (i,)),
            pl.BlockSpec((rows,), lambda i: (i,)),
        ],
        out_specs=pl.BlockSpec((rows,), lambda i: (i,)),
        out_shape=jax.ShapeDtypeStruct((_B,), jnp.float32),
    )(theta_g, knowledge, slip_g, guess_g)
    return out
